# Initial kernel scaffold; baseline (speedup 1.0000x reference)
import functools

import jax
import jax.numpy as jnp
from jax import lax
from jax.experimental import pallas as pl
from jax.experimental.pallas import tpu as pltpu
from jax.experimental.pallas import tpu_sc as plsc

_EMB_SIZE = 256
_NUM_NODES = 100000
_TABLE_SEED = 42
_BATCH = 50000

_NC, _NS, _L = 2, 16, 16
_NW = _NC * _NS
_CHUNK = 80
_NCHUNKS = _BATCH // _CHUNK
_ITERS = -(-_NCHUNKS // _NW)


@functools.cache
def _node_embs():
    key = jax.random.key(_TABLE_SEED)
    tab = jax.random.normal(key, (_NUM_NODES, _EMB_SIZE), dtype=jnp.float32)
    return tab / jnp.sqrt(jnp.float32(_EMB_SIZE))


def _sc_kernel(table_hbm, idx_hbm, att_hbm, out_hbm, idx_v, rows_v, att_v):
    wid = lax.axis_index("s") * _NC + lax.axis_index("c")

    @pl.loop(0, _ITERS)
    def _(j):
        chunk = j * _NW + wid

        @pl.when(chunk < _NCHUNKS)
        def _():
            base = chunk * _CHUNK
            pltpu.sync_copy(idx_hbm.at[pl.ds(base, _CHUNK)], idx_v)
            pltpu.sync_copy(table_hbm.at[idx_v], rows_v)
            pltpu.sync_copy(att_hbm.at[idx_v], att_v)

            @pl.loop(0, _CHUNK)
            def _(r):
                a = att_v[r]
                for c in range(_EMB_SIZE // _L):
                    sl = pl.ds(c * _L, _L)
                    rows_v[r, sl] = rows_v[r, sl] * a

            pltpu.sync_copy(rows_v, out_hbm.at[pl.ds(base, _CHUNK)])


def kernel(node_idx, attenuation):
    table = _node_embs()
    mesh = plsc.VectorSubcoreMesh(core_axis_name="c", subcore_axis_name="s")
    k = pl.kernel(
        _sc_kernel,
        out_type=jax.ShapeDtypeStruct((_BATCH, _EMB_SIZE), jnp.float32),
        mesh=mesh,
        scratch_types=[
            pltpu.VMEM((_CHUNK,), jnp.int32),
            pltpu.VMEM((_CHUNK, _EMB_SIZE), jnp.float32),
            pltpu.VMEM((_CHUNK,), jnp.float32),
        ],
    )
    return k(table, node_idx, attenuation)


# speedup vs baseline: 1.0323x; 1.0323x over previous
import functools

import jax
import jax.numpy as jnp
from jax import lax
from jax.experimental import pallas as pl
from jax.experimental.pallas import tpu as pltpu
from jax.experimental.pallas import tpu_sc as plsc

_EMB_SIZE = 256
_NUM_NODES = 100000
_TABLE_SEED = 42
_BATCH = 50000

_NC, _NS, _L = 2, 16, 16
_NW = _NC * _NS
_CHUNK = 80
_NCHUNKS = _BATCH // _CHUNK
_ITERS = -(-_NCHUNKS // _NW)


@functools.cache
def _node_embs():
    key = jax.random.key(_TABLE_SEED)
    tab = jax.random.normal(key, (_NUM_NODES, _EMB_SIZE), dtype=jnp.float32)
    return tab / jnp.sqrt(jnp.float32(_EMB_SIZE))


def _sc_kernel(table_hbm, idx_hbm, att_hbm, out_hbm, idx_v, rows_v, att_v):
    wid = lax.axis_index("s") * _NC + lax.axis_index("c")

    @pl.loop(0, _ITERS)
    def _(j):
        chunk = j * _NW + wid

        @pl.when(chunk < _NCHUNKS)
        def _():
            base = chunk * _CHUNK
            pltpu.sync_copy(idx_hbm.at[pl.ds(base, _CHUNK)], idx_v)
            pltpu.sync_copy(table_hbm.at[idx_v], rows_v)
            pltpu.sync_copy(att_hbm.at[idx_v], att_v)

            @pl.loop(0, _CHUNK // _L)
            def _(g):
                av = att_v[pl.ds(g * _L, _L)]
                for k in range(_L):
                    a = av[k]
                    r = g * _L + k
                    for c in range(_EMB_SIZE // _L):
                        sl = pl.ds(c * _L, _L)
                        rows_v[r, sl] = rows_v[r, sl] * a

            pltpu.sync_copy(rows_v, out_hbm.at[pl.ds(base, _CHUNK)])


def kernel(node_idx, attenuation):
    table = _node_embs()
    mesh = plsc.VectorSubcoreMesh(core_axis_name="c", subcore_axis_name="s")
    k = pl.kernel(
        _sc_kernel,
        out_type=jax.ShapeDtypeStruct((_BATCH, _EMB_SIZE), jnp.float32),
        mesh=mesh,
        scratch_types=[
            pltpu.VMEM((_CHUNK,), jnp.int32),
            pltpu.VMEM((_CHUNK, _EMB_SIZE), jnp.float32),
            pltpu.VMEM((_CHUNK,), jnp.float32),
        ],
    )
    return k(table, node_idx, attenuation)


# speedup vs baseline: 1.0937x; 1.0595x over previous
import functools

import jax
import jax.numpy as jnp
from jax import lax
from jax.experimental import pallas as pl
from jax.experimental.pallas import tpu as pltpu
from jax.experimental.pallas import tpu_sc as plsc

_EMB_SIZE = 256
_NUM_NODES = 100000
_TABLE_SEED = 42
_BATCH = 50000

_NC, _NS, _L = 2, 16, 16
_NW = _NC * _NS
_CHUNK = 200
_NCHUNKS = _BATCH // _CHUNK
_SLOTS = -(-_NCHUNKS // _NW)
_ROWS_W = _SLOTS * _CHUNK
_GROUPS = _CHUNK // _L


@functools.cache
def _node_embs():
    key = jax.random.key(_TABLE_SEED)
    tab = jax.random.normal(key, (_NUM_NODES, _EMB_SIZE), dtype=jnp.float32)
    return tab / jnp.sqrt(jnp.float32(_EMB_SIZE))


def _scale_chunk(rows_v, att_all, k):
    def scale_group(att_off, row_off, lo):
        av = att_all[pl.ds(att_off, _L)]
        for i in range(lo, _L):
            a = av[i]
            r = row_off + i
            for c in range(_EMB_SIZE // _L):
                sl = pl.ds(c * _L, _L)
                rows_v[r, sl] = rows_v[r, sl] * a

    @pl.loop(0, _GROUPS)
    def _(g):
        scale_group(k * _CHUNK + g * _L, g * _L, 0)

    scale_group(k * _CHUNK + _CHUNK - _L, _CHUNK - _L, 8)


def _sc_kernel(table_hbm, idx_hbm, att_hbm, out_hbm,
               idx_all, att_all, rows0, rows1,
               sem_idx, sem_att, sem_g0, sem_g1, sem_s0, sem_s1):
    wid = lax.axis_index("s") * _NC + lax.axis_index("c")
    bases = [jnp.minimum(wid + k * _NW, _NCHUNKS - 1) * _CHUNK
             for k in range(_SLOTS)]
    rows = (rows0, rows1)
    sem_g = (sem_g0, sem_g1)
    sem_s = (sem_s0, sem_s1)

    idx_cps = []
    for k in range(_SLOTS):
        idx_cps.append(pltpu.async_copy(
            idx_hbm.at[pl.ds(bases[k], _CHUNK)],
            idx_all.at[pl.ds(k * _CHUNK, _CHUNK)], sem_idx))
    for cp in idx_cps:
        cp.wait()

    att_cp = pltpu.async_copy(att_hbm.at[idx_all], att_all, sem_att)

    def gather(k):
        b = k & 1
        return pltpu.async_copy(
            table_hbm.at[idx_all.at[pl.ds(k * _CHUNK, _CHUNK)]],
            rows[b], sem_g[b])

    g_cp = {0: gather(0)}
    s_cp = {}
    att_cp.wait()
    for k in range(_SLOTS):
        b = k & 1
        if k + 1 < _SLOTS:
            if k - 1 >= 0:
                s_cp[k - 1].wait()
            g_cp[k + 1] = gather(k + 1)
        g_cp[k].wait()
        _scale_chunk(rows[b], att_all, k)
        s_cp[k] = pltpu.async_copy(
            rows[b], out_hbm.at[pl.ds(bases[k], _CHUNK)], sem_s[b])
    s_cp[_SLOTS - 2].wait()
    s_cp[_SLOTS - 1].wait()


def kernel(node_idx, attenuation):
    table = _node_embs()
    mesh = plsc.VectorSubcoreMesh(core_axis_name="c", subcore_axis_name="s")
    k = pl.kernel(
        _sc_kernel,
        out_type=jax.ShapeDtypeStruct((_BATCH, _EMB_SIZE), jnp.float32),
        mesh=mesh,
        scratch_types=[
            pltpu.VMEM((_ROWS_W,), jnp.int32),
            pltpu.VMEM((_ROWS_W,), jnp.float32),
            pltpu.VMEM((_CHUNK, _EMB_SIZE), jnp.float32),
            pltpu.VMEM((_CHUNK, _EMB_SIZE), jnp.float32),
            pltpu.SemaphoreType.DMA,
            pltpu.SemaphoreType.DMA,
            pltpu.SemaphoreType.DMA,
            pltpu.SemaphoreType.DMA,
            pltpu.SemaphoreType.DMA,
            pltpu.SemaphoreType.DMA,
        ],
    )
    return k(table, node_idx, attenuation)


# speedup vs baseline: 3.9229x; 3.5868x over previous
import dataclasses

import jax
import jax.numpy as jnp
from jax import lax
from jax.experimental import pallas as pl
from jax.experimental.pallas import tpu as pltpu
from jax.experimental.pallas import tpu_sc as plsc

_EMB_SIZE = 256
_NUM_NODES = 100000
_TABLE_SEED = 42
_BATCH = 50000

_NC, _NS, _L = 2, 16, 16
_NW = _NC * _NS
_HW = 128
_CHUNK = 200
_NCHUNKS = _BATCH // _CHUNK
_SLOTS = -(-_NCHUNKS // _NW)
_ROWS_W = _SLOTS * _CHUNK
_GROUPS = _CHUNK // _L
_PAIR = 2 * _CHUNK


def _make_node_embs():
    key = jax.random.key(_TABLE_SEED)
    tab = jax.random.normal(key, (_NUM_NODES, _EMB_SIZE), dtype=jnp.float32)
    return tab / jnp.sqrt(jnp.float32(_EMB_SIZE))


_NODE_EMBS_HALF = _make_node_embs().reshape(_NUM_NODES * 2, _HW)

_PATTERNS = jnp.concatenate([
    jnp.repeat(jnp.arange(8, dtype=jnp.int32), 2),
    jnp.tile(jnp.arange(2, dtype=jnp.int32), 8),
])


def _scale_chunk(rows_v, att_all, k):
    def scale_group(att_off, row_off, lo):
        av = att_all[pl.ds(att_off, _L)]
        for i in range(lo, _L):
            a = av[i]
            r2 = (row_off + i) * 2
            for half in range(2):
                for c in range(_HW // _L):
                    sl = pl.ds(c * _L, _L)
                    rows_v[r2 + half, sl] = rows_v[r2 + half, sl] * a

    @pl.loop(0, _GROUPS)
    def _(g):
        scale_group(k * _CHUNK + g * _L, g * _L, 0)

    scale_group(k * _CHUNK + _CHUNK - _L, _CHUNK - _L, 8)


def _sc_kernel(table_hbm, idx_hbm, att_hbm, pat_hbm, out_hbm,
               idx_all, att_all, pat_v, pair0, pair1, rows0, rows1,
               sem_idx, sem_att, sem_g0, sem_g1, sem_s0, sem_s1):
    wid = lax.axis_index("s") * _NC + lax.axis_index("c")
    pltpu.sync_copy(pat_hbm, pat_v)
    half = pat_v[pl.ds(0, _L)]
    parity = pat_v[pl.ds(_L, _L)]

    def build_pairs(pair_v, k):
        @pl.loop(0, _PAIR // _L)
        def _(g):
            src = plsc.load_gather(idx_all, [k * _CHUNK + g * 8 + half])
            pair_v[pl.ds(g * _L, _L)] = src * 2 + parity

    chunks = [jnp.minimum(wid + k * _NW, _NCHUNKS - 1) for k in range(_SLOTS)]
    pairs = (pair0, pair1)
    rows = (rows0, rows1)
    sem_g = (sem_g0, sem_g1)
    sem_s = (sem_s0, sem_s1)

    idx_cps = []
    for k in range(_SLOTS):
        idx_cps.append(pltpu.async_copy(
            idx_hbm.at[pl.ds(chunks[k] * _CHUNK, _CHUNK)],
            idx_all.at[pl.ds(k * _CHUNK, _CHUNK)], sem_idx))
    for cp in idx_cps:
        cp.wait()

    att_cp = pltpu.async_copy(att_hbm.at[idx_all], att_all, sem_att)

    def gather(k):
        b = k & 1
        build_pairs(pairs[b], k)
        return pltpu.async_copy(table_hbm.at[pairs[b]], rows[b], sem_g[b])

    g_cp = {0: gather(0)}
    s_cp = {}
    att_cp.wait()
    for k in range(_SLOTS):
        b = k & 1
        if k + 1 < _SLOTS:
            if k - 1 >= 0:
                s_cp[k - 1].wait()
            g_cp[k + 1] = gather(k + 1)
        g_cp[k].wait()
        _scale_chunk(rows[b], att_all, k)
        s_cp[k] = pltpu.async_copy(
            rows[b], out_hbm.at[pl.ds(chunks[k] * _PAIR, _PAIR)], sem_s[b])
    s_cp[_SLOTS - 2].wait()
    s_cp[_SLOTS - 1].wait()


def kernel(node_idx, attenuation):
    cp = pltpu.CompilerParams()
    if "needs_layout_passes" in pltpu.CompilerParams.__dataclass_fields__:
        cp = dataclasses.replace(cp, needs_layout_passes=False)
    mesh = plsc.VectorSubcoreMesh(core_axis_name="c", subcore_axis_name="s")
    k = pl.kernel(
        _sc_kernel,
        compiler_params=cp,
        out_type=jax.ShapeDtypeStruct((_BATCH * 2, _HW), jnp.float32),
        mesh=mesh,
        scratch_types=[
            pltpu.VMEM((_ROWS_W,), jnp.int32),
            pltpu.VMEM((_ROWS_W,), jnp.float32),
            pltpu.VMEM((2 * _L,), jnp.int32),
            pltpu.VMEM((_PAIR,), jnp.int32),
            pltpu.VMEM((_PAIR,), jnp.int32),
            pltpu.VMEM((_PAIR, _HW), jnp.float32),
            pltpu.VMEM((_PAIR, _HW), jnp.float32),
            pltpu.SemaphoreType.DMA,
            pltpu.SemaphoreType.DMA,
            pltpu.SemaphoreType.DMA,
            pltpu.SemaphoreType.DMA,
            pltpu.SemaphoreType.DMA,
            pltpu.SemaphoreType.DMA,
        ],
    )
    out_half = k(_NODE_EMBS_HALF, node_idx, attenuation, _PATTERNS)
    return out_half.reshape(_BATCH, _EMB_SIZE)


# speedup vs baseline: 5.6793x; 1.4477x over previous
import jax
import jax.numpy as jnp
from jax import lax
from jax.experimental import pallas as pl
from jax.experimental.pallas import tpu as pltpu
from jax.experimental.pallas import tpu_sc as plsc

_EMB_SIZE = 256
_NUM_NODES = 100000
_TABLE_SEED = 42
_BATCH = 50000

_NC, _NS, _L = 2, 16, 16
_NW = _NC * _NS
_CHUNK = 200
_NCHUNKS = _BATCH // _CHUNK
_SLOTS = -(-_NCHUNKS // _NW)
_ROWS_W = _SLOTS * _CHUNK
_GROUPS = _CHUNK // _L


def _make_node_embs():
    key = jax.random.key(_TABLE_SEED)
    tab = jax.random.normal(key, (_NUM_NODES, _EMB_SIZE), dtype=jnp.float32)
    return tab / jnp.sqrt(jnp.float32(_EMB_SIZE))


_NODE_EMBS = _make_node_embs()


def _scale_chunk(rows_v, att_all, k):
    def scale_group(att_off, row_off, lo):
        av = att_all[pl.ds(att_off, _L)]
        for i in range(lo, _L):
            a = av[i]
            r = row_off + i
            for c in range(_EMB_SIZE // _L):
                sl = pl.ds(c * _L, _L)
                rows_v[r, sl] = rows_v[r, sl] * a

    @pl.loop(0, _GROUPS)
    def _(g):
        scale_group(k * _CHUNK + g * _L, g * _L, 0)

    scale_group(k * _CHUNK + _CHUNK - _L, _CHUNK - _L, 8)


def _sc_kernel(table_hbm, idx_hbm, att_hbm, out_hbm,
               idx_all, att_all, rows0, rows1,
               sem_idx, sem_att, sem_g0, sem_g1, sem_s0, sem_s1):
    wid = lax.axis_index("s") * _NC + lax.axis_index("c")
    bases = [jnp.minimum(wid + k * _NW, _NCHUNKS - 1) * _CHUNK
             for k in range(_SLOTS)]
    rows = (rows0, rows1)
    sem_g = (sem_g0, sem_g1)
    sem_s = (sem_s0, sem_s1)

    idx_cps = []
    for k in range(_SLOTS):
        idx_cps.append(pltpu.async_copy(
            idx_hbm.at[pl.ds(bases[k], _CHUNK)],
            idx_all.at[pl.ds(k * _CHUNK, _CHUNK)], sem_idx))
    for cp in idx_cps:
        cp.wait()

    att_cp = pltpu.async_copy(att_hbm.at[idx_all], att_all, sem_att)

    def gather(k):
        b = k & 1
        return pltpu.async_copy(
            table_hbm.at[idx_all.at[pl.ds(k * _CHUNK, _CHUNK)]],
            rows[b], sem_g[b])

    g_cp = {0: gather(0)}
    s_cp = {}
    att_cp.wait()
    for k in range(_SLOTS):
        b = k & 1
        if k + 1 < _SLOTS:
            if k - 1 >= 0:
                s_cp[k - 1].wait()
            g_cp[k + 1] = gather(k + 1)
        g_cp[k].wait()
        _scale_chunk(rows[b], att_all, k)
        s_cp[k] = pltpu.async_copy(
            rows[b], out_hbm.at[pl.ds(bases[k], _CHUNK)], sem_s[b])
    s_cp[_SLOTS - 2].wait()
    s_cp[_SLOTS - 1].wait()


def kernel(node_idx, attenuation):
    table = _NODE_EMBS
    mesh = plsc.VectorSubcoreMesh(core_axis_name="c", subcore_axis_name="s")
    k = pl.kernel(
        _sc_kernel,
        out_type=jax.ShapeDtypeStruct((_BATCH, _EMB_SIZE), jnp.float32),
        mesh=mesh,
        scratch_types=[
            pltpu.VMEM((_ROWS_W,), jnp.int32),
            pltpu.VMEM((_ROWS_W,), jnp.float32),
            pltpu.VMEM((_CHUNK, _EMB_SIZE), jnp.float32),
            pltpu.VMEM((_CHUNK, _EMB_SIZE), jnp.float32),
            pltpu.SemaphoreType.DMA,
            pltpu.SemaphoreType.DMA,
            pltpu.SemaphoreType.DMA,
            pltpu.SemaphoreType.DMA,
            pltpu.SemaphoreType.DMA,
            pltpu.SemaphoreType.DMA,
        ],
    )
    return k(table, node_idx, attenuation)
